# Initial kernel scaffold; baseline (speedup 1.0000x reference)
import jax
import jax.numpy as jnp
from jax.experimental import pallas as pl


def kernel(x, edge_index, W1, as1, ad1, b1, Wl, bl, W2, as2, ad2, b2, Wm, bm):
    raise NotImplementedError("write your pallas kernel here")


# speedup vs baseline: 32.2760x; 32.2760x over previous
import functools

import jax
import jax.numpy as jnp
from jax import lax
from jax.experimental import pallas as pl
from jax.experimental.pallas import tpu as pltpu
from jax.experimental.pallas import tpu_sc as plsc

_NC = 2
_NS = 16
_NW = _NC * _NS
_LANES = 16
_CH = 80


def _tc_dense1(x, W1, as2d, ad2d):
    N, din = x.shape
    dh = W1.shape[1]
    B = 1000

    def body(x_ref, w_ref, a_ref, b_ref, h_ref, av_ref, bv_ref):
        h = jnp.dot(x_ref[...], w_ref[...], preferred_element_type=jnp.float32)
        h_ref[...] = h
        av_ref[...] = jnp.dot(h, a_ref[...], preferred_element_type=jnp.float32)
        bv_ref[...] = jnp.dot(h, b_ref[...], preferred_element_type=jnp.float32)

    return pl.pallas_call(
        body,
        grid=(N // B,),
        in_specs=[
            pl.BlockSpec((B, din), lambda i: (i, 0)),
            pl.BlockSpec((din, dh), lambda i: (0, 0)),
            pl.BlockSpec((dh, 1), lambda i: (0, 0)),
            pl.BlockSpec((dh, 1), lambda i: (0, 0)),
        ],
        out_specs=[
            pl.BlockSpec((B, dh), lambda i: (i, 0)),
            pl.BlockSpec((B, 1), lambda i: (i, 0)),
            pl.BlockSpec((B, 1), lambda i: (i, 0)),
        ],
        out_shape=[
            jax.ShapeDtypeStruct((N, dh), jnp.float32),
            jax.ShapeDtypeStruct((N, 1), jnp.float32),
            jax.ShapeDtypeStruct((N, 1), jnp.float32),
        ],
    )(x, W1, as2d, ad2d)


def _tc_mid(acc0, acc1, den0, den1, b1, Wl, bl, W2, as2d, ad2d):
    N, dh = acc0.shape
    dl = Wl.shape[1]
    d2 = W2.shape[1]
    B = 1000

    def body(a0, a1, d0, d1, b1r, wl, blr, w2, ar, br, h2_ref, av_ref, bv_ref):
        den = d0[...] + d1[...] + 1e-16
        m = (a0[...] + a1[...]) / den + b1r[...]
        m = jnp.maximum(m, 0.0)
        hl = jnp.dot(m, wl[...], preferred_element_type=jnp.float32) + blr[...]
        h2 = jnp.dot(hl, w2[...], preferred_element_type=jnp.float32)
        h2_ref[...] = h2
        av_ref[...] = jnp.dot(h2, ar[...], preferred_element_type=jnp.float32)
        bv_ref[...] = jnp.dot(h2, br[...], preferred_element_type=jnp.float32)

    row = lambda i: (i, 0)
    whole = lambda i: (0, 0)
    return pl.pallas_call(
        body,
        grid=(N // B,),
        in_specs=[
            pl.BlockSpec((B, dh), row),
            pl.BlockSpec((B, dh), row),
            pl.BlockSpec((B, 1), row),
            pl.BlockSpec((B, 1), row),
            pl.BlockSpec((1, dh), whole),
            pl.BlockSpec((dh, dl), whole),
            pl.BlockSpec((1, dl), whole),
            pl.BlockSpec((dl, d2), whole),
            pl.BlockSpec((d2, 1), whole),
            pl.BlockSpec((d2, 1), whole),
        ],
        out_specs=[
            pl.BlockSpec((B, d2), row),
            pl.BlockSpec((B, 1), row),
            pl.BlockSpec((B, 1), row),
        ],
        out_shape=[
            jax.ShapeDtypeStruct((N, d2), jnp.float32),
            jax.ShapeDtypeStruct((N, 1), jnp.float32),
            jax.ShapeDtypeStruct((N, 1), jnp.float32),
        ],
    )(acc0, acc1, den0, den1, b1, Wl, bl, W2, as2d, ad2d)


def _tc_final(acc0, acc1, den0, den1, b2, Wm, bm):
    N, d2 = acc0.shape
    B = 1000

    def body(a0, a1, d0, d1, b2r, wm, bmr, out_ref):
        den = d0[...] + d1[...] + 1e-16
        g = (a0[...] + a1[...]) / den + b2r[...]
        g = jnp.maximum(g, 0.0)
        out_ref[...] = (
            jnp.dot(g, wm[...], preferred_element_type=jnp.float32) + bmr[...]
        )

    row = lambda i: (i, 0)
    whole = lambda i: (0, 0)
    return pl.pallas_call(
        body,
        grid=(N // B,),
        in_specs=[
            pl.BlockSpec((B, d2), row),
            pl.BlockSpec((B, d2), row),
            pl.BlockSpec((B, 1), row),
            pl.BlockSpec((B, 1), row),
            pl.BlockSpec((1, d2), whole),
            pl.BlockSpec((d2, 1), whole),
            pl.BlockSpec((1, 1), whole),
        ],
        out_specs=pl.BlockSpec((B, 1), row),
        out_shape=jax.ShapeDtypeStruct((N, 1), jnp.float32),
    )(acc0, acc1, den0, den1, b2, Wm, bm)


@functools.lru_cache(maxsize=None)
def _make_sc_edge_kernel(N, D, E):
    EPW = E // _NW
    NCHUNK = EPW // _CH
    ROWS_T = ((N + _NS - 1) // _NS + 7) // 8 * 8
    NP = ROWS_T * _NS
    DCH = 8
    NB = 5
    NBLK = NCHUNK // NB
    KV = _CH // _LANES
    TD = (ROWS_T + _LANES - 1) // _LANES * _LANES
    mesh = plsc.VectorSubcoreMesh(core_axis_name="c", subcore_axis_name="s")

    @functools.partial(
        pl.kernel,
        out_type=(
            jax.ShapeDtypeStruct((_NC * NP, D), jnp.float32),
            jax.ShapeDtypeStruct((_NC * NP,), jnp.float32),
        ),
        mesh=mesh,
        compiler_params=pltpu.CompilerParams(
            needs_layout_passes=False, use_tc_tiling_on_sc=False),
        scratch_types=[
            pltpu.VMEM((N,), jnp.float32),
            pltpu.VMEM((N,), jnp.float32),
            pltpu.VMEM((NB, _CH), jnp.int32),
            pltpu.VMEM((NB, _CH), jnp.int32),
            pltpu.VMEM((_CH, D), jnp.float32),
            pltpu.VMEM((_CH,), jnp.float32),
            pltpu.VMEM((DCH, D), jnp.float32),
            pltpu.VMEM((TD,), jnp.float32),
            pltpu.VMEM_SHARED((NP, D), jnp.float32),
            pltpu.VMEM_SHARED((NP,), jnp.float32),
            pltpu.SemaphoreType.DMA,
        ],
    )
    def k(h_hbm, av_hbm, bv_hbm, src_hbm, dst_hbm, acc_out, den_out,
          av_v, bv_v, src_v, dst_v, rows_v, ex_v, tmp_v, tmpd_v,
          acc_sh, den_sh, sem):
        c = lax.axis_index("c")
        s = lax.axis_index("s")
        w = c * _NS + s
        pltpu.sync_copy(av_hbm, av_v)
        pltpu.sync_copy(bv_hbm, bv_v)

        def zrow(r, _):
            for i in range(D // _LANES):
                tmp_v[r, pl.ds(i * _LANES, _LANES)] = jnp.zeros(
                    (_LANES,), jnp.float32)
            return 0

        lax.fori_loop(0, DCH, zrow, 0)

        def zden(i, _):
            tmpd_v[pl.ds(i * _LANES, _LANES)] = jnp.zeros((_LANES,), jnp.float32)
            return 0

        lax.fori_loop(0, TD // _LANES, zden, 0)
        row0 = s * ROWS_T

        def zacc(i, _):
            pltpu.sync_copy(tmp_v, acc_sh.at[pl.ds(row0 + i * DCH, DCH)])
            return 0

        lax.fori_loop(0, ROWS_T // DCH, zacc, 0)
        pltpu.sync_copy(tmpd_v.at[pl.ds(0, ROWS_T)], den_sh.at[pl.ds(row0, ROWS_T)])
        plsc.subcore_barrier()

        def chunk(j, _):
            cp = pltpu.async_copy(h_hbm.at[src_v.at[j]], rows_v, sem)
            for kk in range(KV):
                sl = pl.ds(kk * _LANES, _LANES)
                si = src_v[j, sl]
                di = dst_v[j, sl]
                e = plsc.load_gather(av_v, [si]) + plsc.load_gather(bv_v, [di])
                e = jnp.where(e > 0.0, e, 0.2 * e)
                ex_v[sl] = jnp.exp(e)
            cp.wait()
            for kk in range(KV):
                exk = ex_v[pl.ds(kk * _LANES, _LANES)]
                for l in range(_LANES):
                    r = kk * _LANES + l
                    ab = jnp.full((_LANES,), exk[l], jnp.float32)
                    for i in range(D // _LANES):
                        sl = pl.ds(i * _LANES, _LANES)
                        rows_v[r, sl] = rows_v[r, sl] * ab
            pltpu.sync_copy(rows_v, acc_sh.at[dst_v.at[j]], add=True)
            pltpu.sync_copy(ex_v, den_sh.at[dst_v.at[j]], add=True)
            return 0

        def block(b, _):
            pltpu.sync_copy(src_hbm.at[w].at[pl.ds(b * NB, NB)], src_v)
            pltpu.sync_copy(dst_hbm.at[w].at[pl.ds(b * NB, NB)], dst_v)
            lax.fori_loop(0, NB, chunk, 0)
            return 0

        lax.fori_loop(0, NBLK, block, 0)
        plsc.subcore_barrier()

        base = c * NP + row0

        def dump(i, _):
            pltpu.sync_copy(acc_sh.at[pl.ds(row0 + i * DCH, DCH)], tmp_v)
            pltpu.sync_copy(tmp_v, acc_out.at[pl.ds(base + i * DCH, DCH)])
            return 0

        lax.fori_loop(0, ROWS_T // DCH, dump, 0)
        pltpu.sync_copy(den_sh.at[pl.ds(row0, ROWS_T)], tmpd_v.at[pl.ds(0, ROWS_T)])
        pltpu.sync_copy(tmpd_v.at[pl.ds(0, ROWS_T)], den_out.at[pl.ds(base, ROWS_T)])

    return k, NP


def kernel(x, edge_index, W1, as1, ad1, b1, Wl, bl, W2, as2, ad2, b2, Wm, bm):
    N, din = x.shape
    E = edge_index.shape[1]
    dh = W1.shape[1]
    d2 = W2.shape[1]
    EPW = E // _NW
    NCHUNK = EPW // _CH

    src3 = edge_index[0].reshape(_NW, NCHUNK, _CH)
    dst3 = edge_index[1].reshape(_NW, NCHUNK, _CH)

    sc1, NP = _make_sc_edge_kernel(N, dh, E)
    sc2, _ = _make_sc_edge_kernel(N, d2, E)

    h1, av1, bv1 = _tc_dense1(x, W1, as1.reshape(-1, 1), ad1.reshape(-1, 1))
    acc1, den1 = sc1(h1, av1.reshape(-1), bv1.reshape(-1), src3, dst3)

    h2, av2, bv2 = _tc_mid(
        acc1[:N], acc1[NP:NP + N],
        den1[:N].reshape(-1, 1), den1[NP:NP + N].reshape(-1, 1),
        b1.reshape(1, -1), Wl, bl.reshape(1, -1), W2,
        as2.reshape(-1, 1), ad2.reshape(-1, 1))
    acc2, den2 = sc2(h2, av2.reshape(-1), bv2.reshape(-1), src3, dst3)

    out = _tc_final(
        acc2[:N], acc2[NP:NP + N],
        den2[:N].reshape(-1, 1), den2[NP:NP + N].reshape(-1, 1),
        b2.reshape(1, -1), Wm, bm.reshape(1, 1))
    return out[:, 0]


# speedup vs baseline: 38.2793x; 1.1860x over previous
import functools

import jax
import jax.numpy as jnp
from jax import lax
from jax.experimental import pallas as pl
from jax.experimental.pallas import tpu as pltpu
from jax.experimental.pallas import tpu_sc as plsc

_NC = 2
_NS = 16
_NW = _NC * _NS
_LANES = 16
_CH = 80


def _tc_dense1(x, W1, as2d, ad2d):
    N, din = x.shape
    dh = W1.shape[1]
    B = 1000

    def body(x_ref, w_ref, a_ref, b_ref, h_ref, av_ref, bv_ref):
        h = jnp.dot(x_ref[...], w_ref[...], precision=lax.Precision.DEFAULT, preferred_element_type=jnp.float32)
        h_ref[...] = h
        av_ref[...] = jnp.sum(h * a_ref[...], axis=1, keepdims=True)
        bv_ref[...] = jnp.sum(h * b_ref[...], axis=1, keepdims=True)

    return pl.pallas_call(
        body,
        grid=(N // B,),
        in_specs=[
            pl.BlockSpec((B, din), lambda i: (i, 0)),
            pl.BlockSpec((din, dh), lambda i: (0, 0)),
            pl.BlockSpec((1, dh), lambda i: (0, 0)),
            pl.BlockSpec((1, dh), lambda i: (0, 0)),
        ],
        out_specs=[
            pl.BlockSpec((B, dh), lambda i: (i, 0)),
            pl.BlockSpec((B, 1), lambda i: (i, 0)),
            pl.BlockSpec((B, 1), lambda i: (i, 0)),
        ],
        out_shape=[
            jax.ShapeDtypeStruct((N, dh), jnp.float32),
            jax.ShapeDtypeStruct((N, 1), jnp.float32),
            jax.ShapeDtypeStruct((N, 1), jnp.float32),
        ],
    )(x, W1, as2d, ad2d)


def _tc_mid(acc0, acc1, den0, den1, b1, Wl, bl, W2, as2d, ad2d):
    N, dh = acc0.shape
    dl = Wl.shape[1]
    d2 = W2.shape[1]
    B = 1000

    def body(a0, a1, d0, d1, b1r, wl, blr, w2, ar, br, h2_ref, av_ref, bv_ref):
        den = d0[...] + d1[...] + 1e-16
        m = (a0[...] + a1[...]) / den + b1r[...]
        m = jnp.maximum(m, 0.0)
        hl = jnp.dot(m, wl[...], precision=lax.Precision.DEFAULT, preferred_element_type=jnp.float32) + blr[...]
        h2 = jnp.dot(hl, w2[...], precision=lax.Precision.DEFAULT, preferred_element_type=jnp.float32)
        h2_ref[...] = h2
        av_ref[...] = jnp.sum(h2 * ar[...], axis=1, keepdims=True)
        bv_ref[...] = jnp.sum(h2 * br[...], axis=1, keepdims=True)

    row = lambda i: (i, 0)
    whole = lambda i: (0, 0)
    return pl.pallas_call(
        body,
        grid=(N // B,),
        in_specs=[
            pl.BlockSpec((B, dh), row),
            pl.BlockSpec((B, dh), row),
            pl.BlockSpec((B, 1), row),
            pl.BlockSpec((B, 1), row),
            pl.BlockSpec((1, dh), whole),
            pl.BlockSpec((dh, dl), whole),
            pl.BlockSpec((1, dl), whole),
            pl.BlockSpec((dl, d2), whole),
            pl.BlockSpec((1, d2), whole),
            pl.BlockSpec((1, d2), whole),
        ],
        out_specs=[
            pl.BlockSpec((B, d2), row),
            pl.BlockSpec((B, 1), row),
            pl.BlockSpec((B, 1), row),
        ],
        out_shape=[
            jax.ShapeDtypeStruct((N, d2), jnp.float32),
            jax.ShapeDtypeStruct((N, 1), jnp.float32),
            jax.ShapeDtypeStruct((N, 1), jnp.float32),
        ],
    )(acc0, acc1, den0, den1, b1, Wl, bl, W2, as2d, ad2d)


def _tc_final(acc0, acc1, den0, den1, b2, Wm, bm):
    N, d2 = acc0.shape
    B = 1000

    def body(a0, a1, d0, d1, b2r, wm, bmr, out_ref):
        den = d0[...] + d1[...] + 1e-16
        g = (a0[...] + a1[...]) / den + b2r[...]
        g = jnp.maximum(g, 0.0)
        out_ref[...] = (
            jnp.dot(g, wm[...], precision=lax.Precision.DEFAULT, preferred_element_type=jnp.float32) + bmr[...]
        )

    row = lambda i: (i, 0)
    whole = lambda i: (0, 0)
    return pl.pallas_call(
        body,
        grid=(N // B,),
        in_specs=[
            pl.BlockSpec((B, d2), row),
            pl.BlockSpec((B, d2), row),
            pl.BlockSpec((B, 1), row),
            pl.BlockSpec((B, 1), row),
            pl.BlockSpec((1, d2), whole),
            pl.BlockSpec((d2, 1), whole),
            pl.BlockSpec((1, 1), whole),
        ],
        out_specs=pl.BlockSpec((B, 1), row),
        out_shape=jax.ShapeDtypeStruct((N, 1), jnp.float32),
    )(acc0, acc1, den0, den1, b2, Wm, bm)


@functools.lru_cache(maxsize=None)
def _make_sc_edge_kernel(N, D, E):
    EPW = E // _NW
    NCHUNK = EPW // _CH
    assert (NCHUNK - 1) % 4 == 0
    NITER = (NCHUNK - 1) // 4
    ROWS_T = ((N + _NS - 1) // _NS + 7) // 8 * 8
    NP = ROWS_T * _NS
    DCH = 8
    KV = _CH // _LANES
    TD = (ROWS_T + _LANES - 1) // _LANES * _LANES
    mesh = plsc.VectorSubcoreMesh(core_axis_name="c", subcore_axis_name="s")

    @functools.partial(
        pl.kernel,
        out_type=(
            jax.ShapeDtypeStruct((_NC * NP, D), jnp.float32),
            jax.ShapeDtypeStruct((_NC * NP,), jnp.float32),
        ),
        mesh=mesh,
        compiler_params=pltpu.CompilerParams(
            needs_layout_passes=False, use_tc_tiling_on_sc=False),
        scratch_types=[
            pltpu.VMEM((2, 2, _CH), jnp.int32),
            pltpu.VMEM((2, 2, _CH), jnp.int32),
            pltpu.VMEM((2, _CH, D), jnp.float32),
            pltpu.VMEM((2, _CH), jnp.float32),
            pltpu.VMEM((2, _CH), jnp.float32),
            pltpu.VMEM((2, _CH), jnp.float32),
            pltpu.VMEM((DCH, D), jnp.float32),
            pltpu.VMEM((TD,), jnp.float32),
            pltpu.VMEM_SHARED((NP, D), jnp.float32),
            pltpu.VMEM_SHARED((NP,), jnp.float32),
            pltpu.SemaphoreType.DMA((2,)),
            pltpu.SemaphoreType.DMA((2,)),
            pltpu.SemaphoreType.DMA((2,)),
        ],
    )
    def k(h_hbm, av_hbm, bv_hbm, src_hbm, dst_hbm, acc_out, den_out,
          src_v, dst_v, rows_v, avg_v, bvg_v, ex_v, tmp_v, tmpd_v,
          acc_sh, den_sh, gat_sem, scat_sem, idx_sem):
        c = lax.axis_index("c")
        s = lax.axis_index("s")
        w = c * _NS + s

        def zrow(r, _):
            for i in range(D // _LANES):
                tmp_v[r, pl.ds(i * _LANES, _LANES)] = jnp.zeros(
                    (_LANES,), jnp.float32)
            return 0

        lax.fori_loop(0, DCH, zrow, 0)

        def zden(i, _):
            tmpd_v[pl.ds(i * _LANES, _LANES)] = jnp.zeros((_LANES,), jnp.float32)
            return 0

        lax.fori_loop(0, TD // _LANES, zden, 0)
        row0 = s * ROWS_T

        def zacc(i, _):
            pltpu.sync_copy(tmp_v, acc_sh.at[pl.ds(row0 + i * DCH, DCH)])
            return 0

        lax.fori_loop(0, ROWS_T // DCH, zacc, 0)
        pltpu.sync_copy(tmpd_v.at[pl.ds(0, ROWS_T)], den_sh.at[pl.ds(row0, ROWS_T)])
        plsc.subcore_barrier()

        def issue_idx(pair, buf):
            sl = pl.ds(2 * pair, 2)
            pltpu.async_copy(src_hbm.at[w].at[sl], src_v.at[buf], idx_sem.at[buf])
            pltpu.async_copy(dst_hbm.at[w].at[sl], dst_v.at[buf], idx_sem.at[buf])

        def wait_idx(buf):
            pltpu.make_async_copy(
                src_hbm.at[w].at[pl.ds(0, 2)], src_v.at[buf],
                idx_sem.at[buf]).wait()
            pltpu.make_async_copy(
                dst_hbm.at[w].at[pl.ds(0, 2)], dst_v.at[buf],
                idx_sem.at[buf]).wait()

        def issue_gathers(p, buf, r):
            sidx = src_v.at[buf].at[r]
            didx = dst_v.at[buf].at[r]
            pltpu.async_copy(h_hbm.at[sidx], rows_v.at[p], gat_sem.at[p])
            pltpu.async_copy(av_hbm.at[sidx], avg_v.at[p], gat_sem.at[p])
            pltpu.async_copy(bv_hbm.at[didx], bvg_v.at[p], gat_sem.at[p])

        def wait_gathers(p):
            dummy = src_v.at[0].at[0]
            pltpu.make_async_copy(h_hbm.at[dummy], rows_v.at[p],
                                  gat_sem.at[p]).wait()
            pltpu.make_async_copy(av_hbm.at[dummy], avg_v.at[p],
                                  gat_sem.at[p]).wait()
            pltpu.make_async_copy(bv_hbm.at[dummy], bvg_v.at[p],
                                  gat_sem.at[p]).wait()

        def issue_scatters(p, buf, r):
            didx = dst_v.at[buf].at[r]
            pltpu.async_copy(rows_v.at[p], acc_sh.at[didx], scat_sem.at[p],
                             add=True)
            pltpu.async_copy(ex_v.at[p], den_sh.at[didx], scat_sem.at[p],
                             add=True)

        def wait_scatters(p):
            dummy = dst_v.at[0].at[0]
            pltpu.make_async_copy(rows_v.at[p], acc_sh.at[dummy],
                                  scat_sem.at[p]).wait()
            pltpu.make_async_copy(ex_v.at[p], den_sh.at[dummy],
                                  scat_sem.at[p]).wait()

        def compute_scale(p):
            for kk in range(KV):
                sl = pl.ds(kk * _LANES, _LANES)
                e = avg_v[p, sl] + bvg_v[p, sl]
                e = jnp.where(e > 0.0, e, 0.2 * e)
                ex_v[p, sl] = jnp.exp(e)
            for kk in range(KV):
                exk = ex_v[p, pl.ds(kk * _LANES, _LANES)]
                for l in range(_LANES):
                    r = kk * _LANES + l
                    ab = jnp.full((_LANES,), exk[l], jnp.float32)
                    for i in range(D // _LANES):
                        sl = pl.ds(i * _LANES, _LANES)
                        rows_v[p, r, sl] = rows_v[p, r, sl] * ab

        def zero_buf(p):
            for kk in range(KV):
                ex_v[p, pl.ds(kk * _LANES, _LANES)] = jnp.zeros(
                    (_LANES,), jnp.float32)

            def zr(r, _):
                for i in range(D // _LANES):
                    rows_v[p, r, pl.ds(i * _LANES, _LANES)] = jnp.zeros(
                        (_LANES,), jnp.float32)
                return 0

            lax.fori_loop(0, _CH, zr, 0)

        pltpu.sync_copy(src_hbm.at[w].at[pl.ds(0, 2)], src_v.at[0])
        pltpu.sync_copy(dst_hbm.at[w].at[pl.ds(0, 2)], dst_v.at[0])
        issue_gathers(0, 0, 0)
        zero_buf(1)
        issue_scatters(1, 0, 0)

        def body(ii, _):
            wait_gathers(0)
            wait_scatters(1)
            issue_idx(2 * ii + 1, 1)
            issue_gathers(1, 0, 1)
            compute_scale(0)
            issue_scatters(0, 0, 0)
            wait_gathers(1)
            wait_idx(1)
            wait_scatters(0)
            issue_gathers(0, 1, 0)
            compute_scale(1)
            issue_scatters(1, 0, 1)
            wait_gathers(0)
            wait_scatters(1)
            issue_idx(2 * ii + 2, 0)
            issue_gathers(1, 1, 1)
            compute_scale(0)
            issue_scatters(0, 1, 0)
            wait_gathers(1)
            wait_idx(0)
            wait_scatters(0)
            issue_gathers(0, 0, 0)
            compute_scale(1)
            issue_scatters(1, 1, 1)
            return 0

        lax.fori_loop(0, NITER, body, 0)

        wait_gathers(0)
        compute_scale(0)
        issue_scatters(0, 0, 0)
        wait_scatters(0)
        wait_scatters(1)
        plsc.subcore_barrier()

        base = c * NP + row0

        def dump(i, _):
            pltpu.sync_copy(acc_sh.at[pl.ds(row0 + i * DCH, DCH)], tmp_v)
            pltpu.sync_copy(tmp_v, acc_out.at[pl.ds(base + i * DCH, DCH)])
            return 0

        lax.fori_loop(0, ROWS_T // DCH, dump, 0)
        pltpu.sync_copy(den_sh.at[pl.ds(row0, ROWS_T)], tmpd_v.at[pl.ds(0, ROWS_T)])
        pltpu.sync_copy(tmpd_v.at[pl.ds(0, ROWS_T)], den_out.at[pl.ds(base, ROWS_T)])

    return k, NP


def kernel(x, edge_index, W1, as1, ad1, b1, Wl, bl, W2, as2, ad2, b2, Wm, bm):
    N, din = x.shape
    E = edge_index.shape[1]
    dh = W1.shape[1]
    d2 = W2.shape[1]
    EPW = E // _NW
    NCHUNK = EPW // _CH

    src3 = jnp.pad(edge_index[0].reshape(_NW, NCHUNK, _CH), ((0, 0), (0, 1), (0, 0)))
    dst3 = jnp.pad(edge_index[1].reshape(_NW, NCHUNK, _CH), ((0, 0), (0, 1), (0, 0)))

    sc1, NP = _make_sc_edge_kernel(N, dh, E)
    sc2, _ = _make_sc_edge_kernel(N, d2, E)

    h1, av1, bv1 = _tc_dense1(x, W1, as1.reshape(1, -1), ad1.reshape(1, -1))
    acc1, den1 = sc1(h1, av1.reshape(-1), bv1.reshape(-1), src3, dst3)

    h2, av2, bv2 = _tc_mid(
        acc1[:N], acc1[NP:NP + N],
        den1[:N].reshape(-1, 1), den1[NP:NP + N].reshape(-1, 1),
        b1.reshape(1, -1), Wl, bl.reshape(1, -1), W2,
        as2.reshape(1, -1), ad2.reshape(1, -1))
    acc2, den2 = sc2(h2, av2.reshape(-1), bv2.reshape(-1), src3, dst3)

    out = _tc_final(
        acc2[:N], acc2[NP:NP + N],
        den2[:N].reshape(-1, 1), den2[NP:NP + N].reshape(-1, 1),
        b2.reshape(1, -1), Wm, bm.reshape(1, 1))
    return out[:, 0]
